# Initial kernel scaffold; baseline (speedup 1.0000x reference)
import jax
import jax.numpy as jnp
from jax.experimental import pallas as pl


def kernel(user_feats, post_content, parent_comment, edge_ucu, edge_comment, edge_publish, params):
    raise NotImplementedError("write your pallas kernel here")


# speedup vs baseline: 12.9240x; 12.9240x over previous
import functools

import jax
import jax.numpy as jnp
from jax import lax
from jax.experimental import pallas as pl
from jax.experimental.pallas import tpu as pltpu
from jax.experimental.pallas import tpu_sc as plsc

N = 10000
E = 160000
IN_DIM = 768
POST_DIM = 1536
H = 256
HEADS = 8
DH = 32
CLS = 9
MHA_HEADS = 4
MHA_DH = H // MHA_HEADS

HALF = 128
W_EDGES = 80
SC_CORES = 2
SC_TILES = 16
EDGES_PER_TILE = E // SC_TILES
WINDOWS = EDGES_PER_TILE // W_EDGES
MSG_ROWS = 5120
MROWS = 5248
DROWS_T = 320
ACC0_ROWS = 5760

_PREC = lax.Precision.HIGHEST


def _dot(a, b):
    return jnp.dot(a, b, precision=_PREC, preferred_element_type=jnp.float32)


def _vgather16(x, idx):
    return lax.gather(
        x, idx[:, None],
        lax.GatherDimensionNumbers(
            offset_dims=(), collapsed_slice_dims=(0,), start_index_map=(0,)),
        slice_sizes=(1,),
        mode=lax.GatherScatterMode.PROMISE_IN_BOUNDS)


def _stage_a_body(uf, post, parent,
                  wu, bu, pw, pb, cw, cb, qkvw, qkvb, outw, outb,
                  fusw, fusb, lng, lnb,
                  gw0, gw1, gw2, al0, al1, al2, ar0, ar1, ar2,
                  gsum, gexp,
                  ctx_o, h_o,
                  fl0, fr0, ea0,
                  fl1, fr1, ea1,
                  fl2, fr2, ea2):
    h = _dot(uf[...], wu[...]) + bu[...]
    h_o[...] = h

    pc = _dot(post[...], pw[...]) + pb[...]
    cc = _dot(parent[...], cw[...]) + cb[...]
    qp = _dot(pc, qkvw[...]) + qkvb[...]
    qc = _dot(cc, qkvw[...]) + qkvb[...]
    q_p, k_p, v_p = qp[:, :H], qp[:, H:2 * H], qp[:, 2 * H:]
    q_c, k_c, v_c = qc[:, :H], qc[:, H:2 * H], qc[:, 2 * H:]

    g = gsum[...]
    inv = 1.0 / jnp.sqrt(jnp.float32(MHA_DH))
    s_pp = _dot(q_p * k_p, g) * inv
    s_pc = _dot(q_p * k_c, g) * inv
    s_cp = _dot(q_c * k_p, g) * inv
    s_cc = _dot(q_c * k_c, g) * inv

    def att_weights(sa, sb):
        m = jnp.maximum(sa, sb)
        ea = jnp.exp(sa - m)
        eb = jnp.exp(sb - m)
        return ea / (ea + eb)

    a_pp = att_weights(s_pp, s_pc)
    a_cp = att_weights(s_cp, s_cc)
    ge = gexp[...]
    A_pp = _dot(a_pp, ge)
    A_cp = _dot(a_cp, ge)
    o_p = A_pp * v_p + (1.0 - A_pp) * v_c
    o_c = A_cp * v_p + (1.0 - A_cp) * v_c
    o_p = _dot(o_p, outw[...]) + outb[...]
    o_c = _dot(o_c, outw[...]) + outb[...]

    f = _dot(o_p, fusw[:H, :]) + _dot(o_c, fusw[H:, :]) + fusb[...]
    mu = jnp.mean(f, axis=-1, keepdims=True)
    var = jnp.mean((f - mu) ** 2, axis=-1, keepdims=True)
    f = (f - mu) * lax.rsqrt(var + 1e-5) * lng[...] + lnb[...]
    ctx_o[...] = jnp.maximum(f, 0.0)

    for gw, al, ar, fl, fr, ea in (
            (gw0, al0, ar0, fl0, fr0, ea0),
            (gw1, al1, ar1, fl1, fr1, ea1),
            (gw2, al2, ar2, fl2, fr2, ea2)):
        feat = _dot(h, gw[...])
        fl[...] = feat[:, :HALF]
        fr[...] = feat[:, HALF:]
        el = _dot(feat, al[...])
        er = _dot(feat, ar[...])
        ea[...] = jnp.concatenate(
            [el, er, jnp.zeros((el.shape[0], 96), jnp.float32)], axis=1)


def _stage_a(uf, post, parent, params, gsum, gexp, spreads):
    bn = 400
    grid = (N // bn,)
    row = lambda i: (i, 0)
    fix = lambda i: (0, 0)

    def rspec(c):
        return pl.BlockSpec((bn, c), row)

    def wspec(shape):
        return pl.BlockSpec(shape, fix)

    p = params
    conv = p['conv']
    gats = p['gat']
    (al0, ar0), (al1, ar1), (al2, ar2) = spreads

    in_specs = [
        rspec(IN_DIM), rspec(POST_DIM), rspec(IN_DIM),
        wspec((IN_DIM, H)), wspec((1, H)),
        wspec((POST_DIM, H)), wspec((1, H)),
        wspec((IN_DIM, H)), wspec((1, H)),
        wspec((H, 3 * H)), wspec((1, 3 * H)),
        wspec((H, H)), wspec((1, H)),
        wspec((2 * H, H)), wspec((1, H)),
        wspec((1, H)), wspec((1, H)),
        wspec((H, H)), wspec((H, H)), wspec((H, H)),
        wspec((H, 16)), wspec((H, 16)), wspec((H, 16)),
        wspec((H, 16)), wspec((H, 16)), wspec((H, 16)),
        wspec((H, MHA_HEADS)), wspec((MHA_HEADS, H)),
    ]
    out_shapes = (
        jax.ShapeDtypeStruct((N, H), jnp.float32),
        jax.ShapeDtypeStruct((N, H), jnp.float32),
    ) + tuple(
        jax.ShapeDtypeStruct(s, jnp.float32)
        for _ in range(3)
        for s in ((N, HALF), (N, HALF), (N, 128))
    )
    out_specs = (rspec(H), rspec(H)) + tuple(
        rspec(c) for _ in range(3) for c in (HALF, HALF, 128))

    args = (
        uf, post, parent,
        p['user_proj']['w'], p['user_proj']['b'][None, :],
        conv['post_w'], conv['post_b'][None, :],
        conv['com_w'], conv['com_b'][None, :],
        conv['qkv_w'], conv['qkv_b'][None, :],
        conv['out_w'], conv['out_b'][None, :],
        conv['fus_w'], conv['fus_b'][None, :],
        conv['ln_g'][None, :], conv['ln_b'][None, :],
        gats[0]['w'], gats[1]['w'], gats[2]['w'],
        al0, al1, al2, ar0, ar1, ar2,
        gsum, gexp,
    )
    return pl.pallas_call(
        _stage_a_body,
        grid=grid,
        in_specs=in_specs,
        out_specs=out_specs,
        out_shape=out_shapes,
    )(*args)


def _gat_edges_sc(feat_stacked, earr, src, dst, q, with_den):
    mesh = plsc.VectorSubcoreMesh(core_axis_name="c", subcore_axis_name="s")
    NH = N // 2
    acc_rows = ACC0_ROWS if with_den else MROWS

    out_type = [jax.ShapeDtypeStruct((SC_CORES, MSG_ROWS, HALF), jnp.float32)]
    scratch = [
        pltpu.VMEM((W_EDGES,), jnp.int32),
        pltpu.VMEM((W_EDGES,), jnp.int32),
        pltpu.VMEM((W_EDGES,), jnp.int32),
        pltpu.VMEM((W_EDGES,), jnp.int32),
        pltpu.VMEM((W_EDGES, 128), jnp.float32),
        pltpu.VMEM((W_EDGES, 128), jnp.float32),
        pltpu.VMEM((W_EDGES, HALF), jnp.float32),
    ]
    if with_den:
        out_type.append(
            jax.ShapeDtypeStruct((SC_CORES, DROWS_T, HALF), jnp.float32))
        scratch.append(pltpu.VMEM((DROWS_T, 128), jnp.float32))
    scratch.append(pltpu.VMEM_SHARED((acc_rows, 128), jnp.float32))

    @functools.partial(
        pl.kernel, out_type=tuple(out_type), mesh=mesh,
        scratch_types=scratch)
    def edge_kernel(feat_h, ea_h, src_h, dst_h, *refs):
        if with_den:
            (outm_h, outd_h, src_v, dst_v, gidx_v, midx_v,
             el_v, er_v, feat_v, den_t, acc) = refs
        else:
            (outm_h, src_v, dst_v, gidx_v, midx_v,
             el_v, er_v, feat_v, acc) = refs
        c = lax.axis_index("c")
        s = lax.axis_index("s")
        zero16 = jnp.zeros((16,), jnp.float32)
        lane = lax.iota(jnp.int32, 16)
        head_mask = lane < 8
        rot_idx = (lane + 8) & 15
        cc = c * NH

        zsrc = den_t if with_den else feat_v
        @pl.loop(0, 40)
        def _(r):
            for j in range(8):
                zsrc[r, pl.ds(j * 16, 16)] = zero16
        if with_den:
            @pl.loop(40, DROWS_T)
            def _(r):
                for j in range(8):
                    den_t[r, pl.ds(j * 16, 16)] = zero16

        per_tile = acc_rows // SC_TILES
        row0 = s * per_tile
        for k in range(per_tile // 40):
            pltpu.sync_copy(zsrc.at[pl.ds(0, 40)],
                            acc.at[pl.ds(row0 + k * 40, 40)])
        if per_tile % 40:
            pltpu.sync_copy(zsrc.at[pl.ds(0, per_tile % 40)],
                            acc.at[pl.ds(row0 + (per_tile // 40) * 40,
                                         per_tile % 40)])
        plsc.subcore_barrier()

        edge0 = s * EDGES_PER_TILE

        @pl.loop(0, WINDOWS)
        def _(w):
            base = edge0 + w * W_EDGES
            pltpu.sync_copy(src_h.at[pl.ds(base, W_EDGES)], src_v)
            pltpu.sync_copy(dst_h.at[pl.ds(base, W_EDGES)], dst_v)

            @pl.loop(0, W_EDGES // 16)
            def _(j):
                sl = pl.ds(j * 16, 16)
                gidx_v[sl] = src_v[sl] + q * N
                nloc = dst_v[sl] - cc
                inr = (nloc >= 0) & (nloc < NH)
                midx_v[sl] = jnp.where(inr, nloc,
                                       NH + 120 + (nloc & 63))

            pltpu.sync_copy(feat_h.at[gidx_v], feat_v)
            pltpu.sync_copy(ea_h.at[src_v], el_v)
            pltpu.sync_copy(ea_h.at[dst_v], er_v)

            @pl.loop(0, W_EDGES // 16)
            def _(jc):
                dchunk = dst_v[pl.ds(jc * 16, 16)] - cc
                for k in range(16):
                    i = jc * 16 + k
                    e = el_v[i, pl.ds(0, 16)] + er_v[i, pl.ds(16, 16)]
                    e = jnp.where(e >= 0.0, e, 0.2 * e)
                    ex = jnp.exp(jnp.minimum(e, 60.0))
                    if with_den:
                        exm = jnp.where(head_mask, ex, 0.0)
                        exr = _vgather16(exm, rot_idx)
                        d = dchunk[k]
                        inr = (d >= 0) & (d < NH)
                        dsub = d & 15
                        r = jnp.where(inr, lax.shift_right_arithmetic(d, 4),
                                      DROWS_T - 1)
                        off = jnp.where(inr,
                                        jnp.where(dsub == 15, 112,
                                                  dsub * 8), 0)
                        exu = jnp.where(dsub == 15, exr, exm)
                        den_t[r, pl.ds(off, 16)] = (
                            den_t[r, pl.ds(off, 16)] + exu)
                    for j in range(4):
                        bsel = jnp.full((16,), 4 * q + j, jnp.int32)
                        b = _vgather16(ex, bsel)
                        feat_v[i, pl.ds(j * 32, 16)] = (
                            feat_v[i, pl.ds(j * 32, 16)] * b)
                        feat_v[i, pl.ds(j * 32 + 16, 16)] = (
                            feat_v[i, pl.ds(j * 32 + 16, 16)] * b)

            pltpu.sync_copy(feat_v, acc.at[midx_v], add=True)

        plsc.subcore_barrier()

        if with_den:
            for k in range(DROWS_T // W_EDGES):
                @pl.loop(0, W_EDGES // 16)
                def _(j):
                    sl = pl.ds(j * 16, 16)
                    midx_v[sl] = lane + (MROWS + k * W_EDGES + j * 16)
                pltpu.sync_copy(den_t.at[pl.ds(k * W_EDGES, W_EDGES)],
                                acc.at[midx_v], add=True)
            plsc.subcore_barrier()

        mrow0 = s * (MSG_ROWS // SC_TILES)
        for k in range(MSG_ROWS // SC_TILES // W_EDGES):
            rows = pl.ds(mrow0 + k * W_EDGES, W_EDGES)
            pltpu.sync_copy(acc.at[rows], el_v)
            pltpu.sync_copy(el_v, outm_h.at[c, rows])
        if with_den:
            @pl.when(s < 8)
            def _():
                rows = pl.ds(s * 40, 40)
                pltpu.sync_copy(acc.at[pl.ds(MROWS + s * 40, 40)],
                                er_v.at[pl.ds(0, 40)])
                pltpu.sync_copy(er_v.at[pl.ds(0, 40)], outd_h.at[c, rows])

    return edge_kernel(feat_stacked, earr, src, dst)


def _stage_c_body(ctx, m00, m01, d0, m10, m11, d1, m20, m21, d2,
                  pj0, pb0, pj1, pb1, pj2, pb2,
                  gwh0, gwc0, gb0, gwh1, gwc1, gb1, gwh2, gwc2, gb2,
                  sw1, sb1, sw2, cw1, cb1, cw2, cb2, dexp,
                  logits_o):
    ctxv = ctx[...]
    de = dexp[...]

    zs = []
    for ma, mb, dn, pj, pb, gwh, gwc, gb in (
            (m00, m01, d0, pj0, pb0, gwh0, gwc0, gb0),
            (m10, m11, d1, pj1, pb1, gwh1, gwc1, gb1),
            (m20, m21, d2, pj2, pb2, gwh2, gwc2, gb2)):
        num = jnp.concatenate([ma[...], mb[...]], axis=1)
        den = _dot(dn[...], de)
        hg = num / (den + 1e-9)
        hg = jnp.where(hg > 0.0, hg, jnp.exp(jnp.minimum(hg, 0.0)) - 1.0)
        hp = _dot(hg, pj[...]) + pb[...]
        gate = jax.nn.sigmoid(_dot(hp, gwh[...]) + _dot(ctxv, gwc[...]) + gb[...])
        zs.append(gate * hp + (1.0 - gate) * ctxv)

    ws = [_dot(jnp.tanh(_dot(z, sw1[...]) + sb1[...]), sw2[...]) for z in zs]
    m = jnp.maximum(jnp.maximum(ws[0], ws[1]), ws[2])
    es = [jnp.exp(wv - m) for wv in ws]
    tot = es[0] + es[1] + es[2]
    hf = (es[0] * zs[0] + es[1] * zs[1] + es[2] * zs[2]) / tot

    hid = jnp.maximum(_dot(hf, cw1[...]) + cb1[...], 0.0)
    logits_o[...] = _dot(hid, cw2[...]) + cb2[...]


def _stage_c(ctx, outs, params):
    bn = 1000
    grid = (N // bn,)
    row = lambda i: (i, 0)
    fix = lambda i: (0, 0)

    def rspec(c):
        return pl.BlockSpec((bn, c), row)

    def wspec(shape):
        return pl.BlockSpec(shape, fix)

    p = params
    gats = p['gat']

    dexp = jnp.repeat(jnp.eye(HEADS, dtype=jnp.float32), DH, axis=1)

    in_specs = [rspec(H)] + [rspec(HALF), rspec(HALF), rspec(HEADS)] * 3 + [
        wspec((H, H)), wspec((1, H)),
        wspec((H, H)), wspec((1, H)),
        wspec((H, H)), wspec((1, H)),
        wspec((H, H)), wspec((H, H)), wspec((1, H)),
        wspec((H, H)), wspec((H, H)), wspec((1, H)),
        wspec((H, H)), wspec((H, H)), wspec((1, H)),
        wspec((H, H)), wspec((1, H)), wspec((H, 1)),
        wspec((H, H)), wspec((1, H)), wspec((H, CLS)), wspec((1, CLS)),
        wspec((HEADS, H)),
    ]

    args = [ctx]
    for m0, m1, dn in outs:
        args.append(m0)
        args.append(m1)
        args.append(dn)
    for g in gats:
        args.append(g['proj_w'])
        args.append(g['proj_b'][None, :])
    for g in gats:
        args.append(g['gate_w'][:H])
        args.append(g['gate_w'][H:])
        args.append(g['gate_b'][None, :])
    args += [
        p['sem']['w1'], p['sem']['b1'][None, :], p['sem']['w2'],
        p['cls']['w1'], p['cls']['b1'][None, :],
        p['cls']['w2'], p['cls']['b2'][None, :],
        dexp,
    ]

    return pl.pallas_call(
        _stage_c_body,
        grid=grid,
        in_specs=in_specs,
        out_specs=pl.BlockSpec((bn, CLS), row),
        out_shape=jax.ShapeDtypeStruct((N, CLS), jnp.float32),
    )(*args)


def _spread_attn(a):
    eye = jnp.eye(HEADS, 16, dtype=jnp.float32)
    return (a[:, :, None] * eye[:, None, :]).reshape(H, 16)


def kernel(user_feats, post_content, parent_comment,
           edge_ucu, edge_comment, edge_publish, params):
    gsum = jnp.repeat(jnp.eye(MHA_HEADS, dtype=jnp.float32), MHA_DH, axis=0)
    gexp = gsum.T
    spreads = [(_spread_attn(g['al']), _spread_attn(g['ar']))
               for g in params['gat']]

    a_out = _stage_a(user_feats, post_content, parent_comment, params,
                     gsum, gexp, spreads)
    ctx = a_out[0]
    path_feats = []
    for i in range(3):
        fl, fr, ea = a_out[2 + 3 * i: 5 + 3 * i]
        path_feats.append((jnp.concatenate([fl, fr], axis=0), ea))

    edges = (edge_ucu, edge_comment, edge_publish)
    outs = []
    for (feat2, ea), ei in zip(path_feats, edges):
        outm0, outd = _gat_edges_sc(feat2, ea, ei[0], ei[1], 0, True)
        res1 = _gat_edges_sc(feat2, ea, ei[0], ei[1], 1, False)
        outm1 = res1[0] if isinstance(res1, (tuple, list)) else res1
        nh = N // 2
        num_l = jnp.concatenate([outm0[0, :nh], outm0[1, :nh]], axis=0)
        num_r = jnp.concatenate([outm1[0, :nh], outm1[1, :nh]], axis=0)
        den8 = jnp.concatenate(
            [outd[0].reshape(DROWS_T * 16, HEADS)[:nh],
             outd[1].reshape(DROWS_T * 16, HEADS)[:nh]], axis=0)
        outs.append((num_l, num_r, den8))

    return _stage_c(ctx, outs, params)


# speedup vs baseline: 17.5534x; 1.3582x over previous
import functools

import jax
import jax.numpy as jnp
from jax import lax
from jax.experimental import pallas as pl
from jax.experimental.pallas import tpu as pltpu
from jax.experimental.pallas import tpu_sc as plsc

N = 10000
E = 160000
IN_DIM = 768
POST_DIM = 1536
H = 256
HEADS = 8
DH = 32
CLS = 9
MHA_HEADS = 4
MHA_DH = H // MHA_HEADS

HALF = 128
W_EDGES = 80
SC_CORES = 2
SC_TILES = 16
EDGES_PER_TILE = E // SC_TILES
WINDOWS = EDGES_PER_TILE // W_EDGES
MSG_ROWS = 5120
MROWS = 5248
DROWS_T = 320
ACC0_ROWS = 5760

_PREC = lax.Precision.HIGHEST


def _dot(a, b):
    return jnp.dot(a, b, precision=_PREC, preferred_element_type=jnp.float32)


def _vgather16(x, idx):
    return lax.gather(
        x, idx[:, None],
        lax.GatherDimensionNumbers(
            offset_dims=(), collapsed_slice_dims=(0,), start_index_map=(0,)),
        slice_sizes=(1,),
        mode=lax.GatherScatterMode.PROMISE_IN_BOUNDS)


def _stage_a_body(uf, post, parent,
                  wu, bu, pw, pb, cw, cb, qkvw, qkvb, outw, outb,
                  fusw, fusb, lng, lnb,
                  gw0, gw1, gw2, al0, al1, al2, ar0, ar1, ar2,
                  gsum, gexp,
                  ctx_o, h_o,
                  fl0, fr0, ea0,
                  fl1, fr1, ea1,
                  fl2, fr2, ea2):
    h = _dot(uf[...], wu[...]) + bu[...]
    h_o[...] = h

    pc = _dot(post[...], pw[...]) + pb[...]
    cc = _dot(parent[...], cw[...]) + cb[...]
    qp = _dot(pc, qkvw[...]) + qkvb[...]
    qc = _dot(cc, qkvw[...]) + qkvb[...]
    q_p, k_p, v_p = qp[:, :H], qp[:, H:2 * H], qp[:, 2 * H:]
    q_c, k_c, v_c = qc[:, :H], qc[:, H:2 * H], qc[:, 2 * H:]

    g = gsum[...]
    inv = 1.0 / jnp.sqrt(jnp.float32(MHA_DH))
    s_pp = _dot(q_p * k_p, g) * inv
    s_pc = _dot(q_p * k_c, g) * inv
    s_cp = _dot(q_c * k_p, g) * inv
    s_cc = _dot(q_c * k_c, g) * inv

    def att_weights(sa, sb):
        m = jnp.maximum(sa, sb)
        ea = jnp.exp(sa - m)
        eb = jnp.exp(sb - m)
        return ea / (ea + eb)

    a_pp = att_weights(s_pp, s_pc)
    a_cp = att_weights(s_cp, s_cc)
    ge = gexp[...]
    A_pp = _dot(a_pp, ge)
    A_cp = _dot(a_cp, ge)
    o_p = A_pp * v_p + (1.0 - A_pp) * v_c
    o_c = A_cp * v_p + (1.0 - A_cp) * v_c
    o_p = _dot(o_p, outw[...]) + outb[...]
    o_c = _dot(o_c, outw[...]) + outb[...]

    f = _dot(o_p, fusw[:H, :]) + _dot(o_c, fusw[H:, :]) + fusb[...]
    mu = jnp.mean(f, axis=-1, keepdims=True)
    var = jnp.mean((f - mu) ** 2, axis=-1, keepdims=True)
    f = (f - mu) * lax.rsqrt(var + 1e-5) * lng[...] + lnb[...]
    ctx_o[...] = jnp.maximum(f, 0.0)

    for gw, al, ar, fl, fr, ea in (
            (gw0, al0, ar0, fl0, fr0, ea0),
            (gw1, al1, ar1, fl1, fr1, ea1),
            (gw2, al2, ar2, fl2, fr2, ea2)):
        feat = _dot(h, gw[...])
        fl[...] = feat[:, :HALF]
        fr[...] = feat[:, HALF:]
        el = _dot(feat, al[...])
        er = _dot(feat, ar[...])
        ea[...] = jnp.concatenate(
            [el, er, jnp.zeros((el.shape[0], 96), jnp.float32)], axis=1)


def _stage_a(uf, post, parent, params, gsum, gexp, spreads):
    bn = 400
    grid = (N // bn,)
    row = lambda i: (i, 0)
    fix = lambda i: (0, 0)

    def rspec(c):
        return pl.BlockSpec((bn, c), row)

    def wspec(shape):
        return pl.BlockSpec(shape, fix)

    p = params
    conv = p['conv']
    gats = p['gat']
    (al0, ar0), (al1, ar1), (al2, ar2) = spreads

    in_specs = [
        rspec(IN_DIM), rspec(POST_DIM), rspec(IN_DIM),
        wspec((IN_DIM, H)), wspec((1, H)),
        wspec((POST_DIM, H)), wspec((1, H)),
        wspec((IN_DIM, H)), wspec((1, H)),
        wspec((H, 3 * H)), wspec((1, 3 * H)),
        wspec((H, H)), wspec((1, H)),
        wspec((2 * H, H)), wspec((1, H)),
        wspec((1, H)), wspec((1, H)),
        wspec((H, H)), wspec((H, H)), wspec((H, H)),
        wspec((H, 16)), wspec((H, 16)), wspec((H, 16)),
        wspec((H, 16)), wspec((H, 16)), wspec((H, 16)),
        wspec((H, MHA_HEADS)), wspec((MHA_HEADS, H)),
    ]
    out_shapes = (
        jax.ShapeDtypeStruct((N, H), jnp.float32),
        jax.ShapeDtypeStruct((N, H), jnp.float32),
    ) + tuple(
        jax.ShapeDtypeStruct(s, jnp.float32)
        for _ in range(3)
        for s in ((N, HALF), (N, HALF), (N, 128))
    )
    out_specs = (rspec(H), rspec(H)) + tuple(
        rspec(c) for _ in range(3) for c in (HALF, HALF, 128))

    args = (
        uf, post, parent,
        p['user_proj']['w'], p['user_proj']['b'][None, :],
        conv['post_w'], conv['post_b'][None, :],
        conv['com_w'], conv['com_b'][None, :],
        conv['qkv_w'], conv['qkv_b'][None, :],
        conv['out_w'], conv['out_b'][None, :],
        conv['fus_w'], conv['fus_b'][None, :],
        conv['ln_g'][None, :], conv['ln_b'][None, :],
        gats[0]['w'], gats[1]['w'], gats[2]['w'],
        al0, al1, al2, ar0, ar1, ar2,
        gsum, gexp,
    )
    return pl.pallas_call(
        _stage_a_body,
        grid=grid,
        in_specs=in_specs,
        out_specs=out_specs,
        out_shape=out_shapes,
    )(*args)


def _gat_edges_sc(feat_stacked, earr, src, dst, q, with_den):
    mesh = plsc.VectorSubcoreMesh(core_axis_name="c", subcore_axis_name="s")
    NH = N // 2
    acc_rows = ACC0_ROWS if with_den else MROWS

    out_type = [jax.ShapeDtypeStruct((SC_CORES, MSG_ROWS, HALF), jnp.float32)]
    scratch = [
        pltpu.VMEM((W_EDGES,), jnp.int32),
        pltpu.VMEM((W_EDGES,), jnp.int32),
        pltpu.VMEM((W_EDGES,), jnp.int32),
        pltpu.VMEM((W_EDGES,), jnp.int32),
        pltpu.VMEM((W_EDGES, 128), jnp.float32),
        pltpu.VMEM((W_EDGES, 128), jnp.float32),
        pltpu.VMEM((W_EDGES, HALF), jnp.float32),
    ]
    if with_den:
        out_type.append(
            jax.ShapeDtypeStruct((SC_CORES, DROWS_T, HALF), jnp.float32))
        scratch.append(pltpu.VMEM((DROWS_T, 128), jnp.float32))
    scratch.append(pltpu.VMEM_SHARED((acc_rows, 128), jnp.float32))
    scratch.append(pltpu.SemaphoreType.DMA)

    @functools.partial(
        pl.kernel, out_type=tuple(out_type), mesh=mesh,
        scratch_types=scratch)
    def edge_kernel(feat_h, ea_h, src_h, dst_h, *refs):
        if with_den:
            (outm_h, outd_h, src_v, dst_v, gidx_v, midx_v,
             el_v, er_v, feat_v, den_t, acc, sem) = refs
        else:
            (outm_h, src_v, dst_v, gidx_v, midx_v,
             el_v, er_v, feat_v, acc, sem) = refs
        c = lax.axis_index("c")
        s = lax.axis_index("s")
        zero16 = jnp.zeros((16,), jnp.float32)
        lane = lax.iota(jnp.int32, 16)
        head_mask = lane < 8
        rot_idx = (lane + 8) & 15
        cc = c * NH

        zsrc = den_t if with_den else feat_v
        @pl.loop(0, 40)
        def _(r):
            for j in range(8):
                zsrc[r, pl.ds(j * 16, 16)] = zero16
        if with_den:
            @pl.loop(40, DROWS_T)
            def _(r):
                for j in range(8):
                    den_t[r, pl.ds(j * 16, 16)] = zero16

        per_tile = acc_rows // SC_TILES
        row0 = s * per_tile
        for k in range(per_tile // 40):
            pltpu.sync_copy(zsrc.at[pl.ds(0, 40)],
                            acc.at[pl.ds(row0 + k * 40, 40)])
        if per_tile % 40:
            pltpu.sync_copy(zsrc.at[pl.ds(0, per_tile % 40)],
                            acc.at[pl.ds(row0 + (per_tile // 40) * 40,
                                         per_tile % 40)])
        plsc.subcore_barrier()

        edge0 = s * EDGES_PER_TILE

        @pl.loop(0, WINDOWS)
        def _(w):
            base = edge0 + w * W_EDGES
            cp1 = pltpu.async_copy(src_h.at[pl.ds(base, W_EDGES)], src_v, sem)
            cp2 = pltpu.async_copy(dst_h.at[pl.ds(base, W_EDGES)], dst_v, sem)
            cp1.wait()
            cp2.wait()

            @pl.loop(0, W_EDGES // 16)
            def _(j):
                sl = pl.ds(j * 16, 16)
                gidx_v[sl] = src_v[sl] + q * N
                nloc = dst_v[sl] - cc
                inr = (nloc >= 0) & (nloc < NH)
                midx_v[sl] = jnp.where(inr, nloc,
                                       NH + 120 + (nloc & 63))

            g1 = pltpu.async_copy(feat_h.at[gidx_v], feat_v, sem)
            g2 = pltpu.async_copy(ea_h.at[src_v], el_v, sem)
            g3 = pltpu.async_copy(ea_h.at[dst_v], er_v, sem)
            g1.wait()
            g2.wait()
            g3.wait()

            @pl.loop(0, W_EDGES // 16)
            def _(jc):
                dchunk = dst_v[pl.ds(jc * 16, 16)] - cc
                for k in range(16):
                    i = jc * 16 + k
                    e = el_v[i, pl.ds(0, 16)] + er_v[i, pl.ds(16, 16)]
                    e = jnp.where(e >= 0.0, e, 0.2 * e)
                    ex = jnp.exp(jnp.minimum(e, 60.0))
                    if with_den:
                        exm = jnp.where(head_mask, ex, 0.0)
                        exr = _vgather16(exm, rot_idx)
                        d = dchunk[k]
                        inr = (d >= 0) & (d < NH)
                        dsub = d & 15
                        r = jnp.where(inr, lax.shift_right_arithmetic(d, 4),
                                      DROWS_T - 1)
                        off = jnp.where(inr,
                                        jnp.where(dsub == 15, 112,
                                                  dsub * 8), 0)
                        exu = jnp.where(dsub == 15, exr, exm)
                        den_t[r, pl.ds(off, 16)] = (
                            den_t[r, pl.ds(off, 16)] + exu)
                    for j in range(4):
                        bsel = jnp.full((16,), 4 * q + j, jnp.int32)
                        b = _vgather16(ex, bsel)
                        feat_v[i, pl.ds(j * 32, 16)] = (
                            feat_v[i, pl.ds(j * 32, 16)] * b)
                        feat_v[i, pl.ds(j * 32 + 16, 16)] = (
                            feat_v[i, pl.ds(j * 32 + 16, 16)] * b)

            pltpu.sync_copy(feat_v, acc.at[midx_v], add=True)

        plsc.subcore_barrier()

        if with_den:
            for k in range(DROWS_T // W_EDGES):
                @pl.loop(0, W_EDGES // 16)
                def _(j):
                    sl = pl.ds(j * 16, 16)
                    midx_v[sl] = lane + (MROWS + k * W_EDGES + j * 16)
                pltpu.sync_copy(den_t.at[pl.ds(k * W_EDGES, W_EDGES)],
                                acc.at[midx_v], add=True)
            plsc.subcore_barrier()

        mrow0 = s * (MSG_ROWS // SC_TILES)
        for k in range(MSG_ROWS // SC_TILES // W_EDGES):
            rows = pl.ds(mrow0 + k * W_EDGES, W_EDGES)
            pltpu.sync_copy(acc.at[rows], el_v)
            pltpu.sync_copy(el_v, outm_h.at[c, rows])
        if with_den:
            @pl.when(s < 8)
            def _():
                rows = pl.ds(s * 40, 40)
                pltpu.sync_copy(acc.at[pl.ds(MROWS + s * 40, 40)],
                                er_v.at[pl.ds(0, 40)])
                pltpu.sync_copy(er_v.at[pl.ds(0, 40)], outd_h.at[c, rows])

    return edge_kernel(feat_stacked, earr, src, dst)


def _stage_c_body(ctx, m00, m01, d0, m10, m11, d1, m20, m21, d2,
                  pj0, pb0, pj1, pb1, pj2, pb2,
                  gwh0, gwc0, gb0, gwh1, gwc1, gb1, gwh2, gwc2, gb2,
                  sw1, sb1, sw2, cw1, cb1, cw2, cb2, dexp,
                  logits_o):
    ctxv = ctx[...]
    de = dexp[...]

    zs = []
    for ma, mb, dn, pj, pb, gwh, gwc, gb in (
            (m00, m01, d0, pj0, pb0, gwh0, gwc0, gb0),
            (m10, m11, d1, pj1, pb1, gwh1, gwc1, gb1),
            (m20, m21, d2, pj2, pb2, gwh2, gwc2, gb2)):
        num = jnp.concatenate([ma[...], mb[...]], axis=1)
        den = _dot(dn[...], de)
        hg = num / (den + 1e-9)
        hg = jnp.where(hg > 0.0, hg, jnp.exp(jnp.minimum(hg, 0.0)) - 1.0)
        hp = _dot(hg, pj[...]) + pb[...]
        gate = jax.nn.sigmoid(_dot(hp, gwh[...]) + _dot(ctxv, gwc[...]) + gb[...])
        zs.append(gate * hp + (1.0 - gate) * ctxv)

    ws = [_dot(jnp.tanh(_dot(z, sw1[...]) + sb1[...]), sw2[...]) for z in zs]
    m = jnp.maximum(jnp.maximum(ws[0], ws[1]), ws[2])
    es = [jnp.exp(wv - m) for wv in ws]
    tot = es[0] + es[1] + es[2]
    hf = (es[0] * zs[0] + es[1] * zs[1] + es[2] * zs[2]) / tot

    hid = jnp.maximum(_dot(hf, cw1[...]) + cb1[...], 0.0)
    logits_o[...] = _dot(hid, cw2[...]) + cb2[...]


def _stage_c(ctx, outs, params):
    bn = 1000
    grid = (N // bn,)
    row = lambda i: (i, 0)
    fix = lambda i: (0, 0)

    def rspec(c):
        return pl.BlockSpec((bn, c), row)

    def wspec(shape):
        return pl.BlockSpec(shape, fix)

    p = params
    gats = p['gat']

    dexp = jnp.repeat(jnp.eye(HEADS, dtype=jnp.float32), DH, axis=1)

    in_specs = [rspec(H)] + [rspec(HALF), rspec(HALF), rspec(HEADS)] * 3 + [
        wspec((H, H)), wspec((1, H)),
        wspec((H, H)), wspec((1, H)),
        wspec((H, H)), wspec((1, H)),
        wspec((H, H)), wspec((H, H)), wspec((1, H)),
        wspec((H, H)), wspec((H, H)), wspec((1, H)),
        wspec((H, H)), wspec((H, H)), wspec((1, H)),
        wspec((H, H)), wspec((1, H)), wspec((H, 1)),
        wspec((H, H)), wspec((1, H)), wspec((H, CLS)), wspec((1, CLS)),
        wspec((HEADS, H)),
    ]

    args = [ctx]
    for m0, m1, dn in outs:
        args.append(m0)
        args.append(m1)
        args.append(dn)
    for g in gats:
        args.append(g['proj_w'])
        args.append(g['proj_b'][None, :])
    for g in gats:
        args.append(g['gate_w'][:H])
        args.append(g['gate_w'][H:])
        args.append(g['gate_b'][None, :])
    args += [
        p['sem']['w1'], p['sem']['b1'][None, :], p['sem']['w2'],
        p['cls']['w1'], p['cls']['b1'][None, :],
        p['cls']['w2'], p['cls']['b2'][None, :],
        dexp,
    ]

    return pl.pallas_call(
        _stage_c_body,
        grid=grid,
        in_specs=in_specs,
        out_specs=pl.BlockSpec((bn, CLS), row),
        out_shape=jax.ShapeDtypeStruct((N, CLS), jnp.float32),
    )(*args)


def _spread_attn(a):
    eye = jnp.eye(HEADS, 16, dtype=jnp.float32)
    return (a[:, :, None] * eye[:, None, :]).reshape(H, 16)


def kernel(user_feats, post_content, parent_comment,
           edge_ucu, edge_comment, edge_publish, params):
    gsum = jnp.repeat(jnp.eye(MHA_HEADS, dtype=jnp.float32), MHA_DH, axis=0)
    gexp = gsum.T
    spreads = [(_spread_attn(g['al']), _spread_attn(g['ar']))
               for g in params['gat']]

    a_out = _stage_a(user_feats, post_content, parent_comment, params,
                     gsum, gexp, spreads)
    ctx = a_out[0]
    path_feats = []
    for i in range(3):
        fl, fr, ea = a_out[2 + 3 * i: 5 + 3 * i]
        path_feats.append((jnp.concatenate([fl, fr], axis=0), ea))

    edges = (edge_ucu, edge_comment, edge_publish)
    outs = []
    for (feat2, ea), ei in zip(path_feats, edges):
        outm0, outd = _gat_edges_sc(feat2, ea, ei[0], ei[1], 0, True)
        res1 = _gat_edges_sc(feat2, ea, ei[0], ei[1], 1, False)
        outm1 = res1[0] if isinstance(res1, (tuple, list)) else res1
        nh = N // 2
        num_l = jnp.concatenate([outm0[0, :nh], outm0[1, :nh]], axis=0)
        num_r = jnp.concatenate([outm1[0, :nh], outm1[1, :nh]], axis=0)
        den8 = jnp.concatenate(
            [outd[0].reshape(DROWS_T * 16, HEADS)[:nh],
             outd[1].reshape(DROWS_T * 16, HEADS)[:nh]], axis=0)
        outs.append((num_l, num_r, den8))

    return _stage_c(ctx, outs, params)


# speedup vs baseline: 17.6198x; 1.0038x over previous
import functools

import jax
import jax.numpy as jnp
from jax import lax
from jax.experimental import pallas as pl
from jax.experimental.pallas import tpu as pltpu
from jax.experimental.pallas import tpu_sc as plsc

N = 10000
E = 160000
IN_DIM = 768
POST_DIM = 1536
H = 256
HEADS = 8
DH = 32
CLS = 9
MHA_HEADS = 4
MHA_DH = H // MHA_HEADS

HALF = 128
W_EDGES = 80
SC_CORES = 2
SC_TILES = 16
EDGES_PER_TILE = E // SC_TILES
WINDOWS = EDGES_PER_TILE // W_EDGES
MSG_ROWS = 5120
MROWS = 5248
DROWS_T = 320
ACC0_ROWS = 5760

_PREC = lax.Precision.HIGHEST


def _dot(a, b):
    return jnp.dot(a, b, precision=_PREC, preferred_element_type=jnp.float32)


def _vgather16(x, idx):
    return lax.gather(
        x, idx[:, None],
        lax.GatherDimensionNumbers(
            offset_dims=(), collapsed_slice_dims=(0,), start_index_map=(0,)),
        slice_sizes=(1,),
        mode=lax.GatherScatterMode.PROMISE_IN_BOUNDS)


def _stage_a_body(uf, post, parent,
                  wu, bu, pw, pb, cw, cb, qkvw, qkvb, outw, outb,
                  fusw, fusb, lng, lnb,
                  gw0, gw1, gw2, al0, al1, al2, ar0, ar1, ar2,
                  gsum, gexp,
                  ctx_o, h_o,
                  fl0, fr0, ea0,
                  fl1, fr1, ea1,
                  fl2, fr2, ea2):
    h = _dot(uf[...], wu[...]) + bu[...]
    h_o[...] = h

    pc = _dot(post[...], pw[...]) + pb[...]
    cc = _dot(parent[...], cw[...]) + cb[...]
    qp = _dot(pc, qkvw[...]) + qkvb[...]
    qc = _dot(cc, qkvw[...]) + qkvb[...]
    q_p, k_p, v_p = qp[:, :H], qp[:, H:2 * H], qp[:, 2 * H:]
    q_c, k_c, v_c = qc[:, :H], qc[:, H:2 * H], qc[:, 2 * H:]

    g = gsum[...]
    inv = 1.0 / jnp.sqrt(jnp.float32(MHA_DH))
    s_pp = _dot(q_p * k_p, g) * inv
    s_pc = _dot(q_p * k_c, g) * inv
    s_cp = _dot(q_c * k_p, g) * inv
    s_cc = _dot(q_c * k_c, g) * inv

    def att_weights(sa, sb):
        m = jnp.maximum(sa, sb)
        ea = jnp.exp(sa - m)
        eb = jnp.exp(sb - m)
        return ea / (ea + eb)

    a_pp = att_weights(s_pp, s_pc)
    a_cp = att_weights(s_cp, s_cc)
    ge = gexp[...]
    A_pp = _dot(a_pp, ge)
    A_cp = _dot(a_cp, ge)
    o_p = A_pp * v_p + (1.0 - A_pp) * v_c
    o_c = A_cp * v_p + (1.0 - A_cp) * v_c
    o_p = _dot(o_p, outw[...]) + outb[...]
    o_c = _dot(o_c, outw[...]) + outb[...]

    f = _dot(o_p, fusw[:H, :]) + _dot(o_c, fusw[H:, :]) + fusb[...]
    mu = jnp.mean(f, axis=-1, keepdims=True)
    var = jnp.mean((f - mu) ** 2, axis=-1, keepdims=True)
    f = (f - mu) * lax.rsqrt(var + 1e-5) * lng[...] + lnb[...]
    ctx_o[...] = jnp.maximum(f, 0.0)

    for gw, al, ar, fl, fr, ea in (
            (gw0, al0, ar0, fl0, fr0, ea0),
            (gw1, al1, ar1, fl1, fr1, ea1),
            (gw2, al2, ar2, fl2, fr2, ea2)):
        feat = _dot(h, gw[...])
        fl[...] = feat[:, :HALF]
        fr[...] = feat[:, HALF:]
        el = _dot(feat, al[...])
        er = _dot(feat, ar[...])
        ea[...] = jnp.concatenate(
            [el, er, jnp.zeros((el.shape[0], 96), jnp.float32)], axis=1)


def _stage_a(uf, post, parent, params, gsum, gexp, spreads):
    bn = 400
    grid = (N // bn,)
    row = lambda i: (i, 0)
    fix = lambda i: (0, 0)

    def rspec(c):
        return pl.BlockSpec((bn, c), row)

    def wspec(shape):
        return pl.BlockSpec(shape, fix)

    p = params
    conv = p['conv']
    gats = p['gat']
    (al0, ar0), (al1, ar1), (al2, ar2) = spreads

    in_specs = [
        rspec(IN_DIM), rspec(POST_DIM), rspec(IN_DIM),
        wspec((IN_DIM, H)), wspec((1, H)),
        wspec((POST_DIM, H)), wspec((1, H)),
        wspec((IN_DIM, H)), wspec((1, H)),
        wspec((H, 3 * H)), wspec((1, 3 * H)),
        wspec((H, H)), wspec((1, H)),
        wspec((2 * H, H)), wspec((1, H)),
        wspec((1, H)), wspec((1, H)),
        wspec((H, H)), wspec((H, H)), wspec((H, H)),
        wspec((H, 16)), wspec((H, 16)), wspec((H, 16)),
        wspec((H, 16)), wspec((H, 16)), wspec((H, 16)),
        wspec((H, MHA_HEADS)), wspec((MHA_HEADS, H)),
    ]
    out_shapes = (
        jax.ShapeDtypeStruct((N, H), jnp.float32),
        jax.ShapeDtypeStruct((N, H), jnp.float32),
    ) + tuple(
        jax.ShapeDtypeStruct(s, jnp.float32)
        for _ in range(3)
        for s in ((N, HALF), (N, HALF), (N, 128))
    )
    out_specs = (rspec(H), rspec(H)) + tuple(
        rspec(c) for _ in range(3) for c in (HALF, HALF, 128))

    args = (
        uf, post, parent,
        p['user_proj']['w'], p['user_proj']['b'][None, :],
        conv['post_w'], conv['post_b'][None, :],
        conv['com_w'], conv['com_b'][None, :],
        conv['qkv_w'], conv['qkv_b'][None, :],
        conv['out_w'], conv['out_b'][None, :],
        conv['fus_w'], conv['fus_b'][None, :],
        conv['ln_g'][None, :], conv['ln_b'][None, :],
        gats[0]['w'], gats[1]['w'], gats[2]['w'],
        al0, al1, al2, ar0, ar1, ar2,
        gsum, gexp,
    )
    return pl.pallas_call(
        _stage_a_body,
        grid=grid,
        in_specs=in_specs,
        out_specs=out_specs,
        out_shape=out_shapes,
    )(*args)


def _gat_edges_sc(feat_stacked, earr, src, dst, q, with_den):
    mesh = plsc.VectorSubcoreMesh(core_axis_name="c", subcore_axis_name="s")
    NH = N // 2
    acc_rows = ACC0_ROWS if with_den else MROWS

    out_type = [jax.ShapeDtypeStruct((SC_CORES, MSG_ROWS, HALF), jnp.float32)]
    scratch = [
        pltpu.VMEM((W_EDGES,), jnp.int32),
        pltpu.VMEM((W_EDGES,), jnp.int32),
        pltpu.VMEM((W_EDGES,), jnp.int32),
        pltpu.VMEM((W_EDGES,), jnp.int32),
        pltpu.VMEM((W_EDGES, 128), jnp.float32),
        pltpu.VMEM((W_EDGES, 128), jnp.float32),
        pltpu.VMEM((W_EDGES, HALF), jnp.float32),
        pltpu.VMEM((W_EDGES, 16), jnp.float32),
    ]
    if with_den:
        out_type.append(
            jax.ShapeDtypeStruct((SC_CORES, DROWS_T, HALF), jnp.float32))
        scratch.append(pltpu.VMEM((DROWS_T, 128), jnp.float32))
    scratch.append(pltpu.VMEM_SHARED((acc_rows, 128), jnp.float32))
    scratch.append(pltpu.SemaphoreType.DMA)

    @functools.partial(
        pl.kernel, out_type=tuple(out_type), mesh=mesh,
        scratch_types=scratch)
    def edge_kernel(feat_h, ea_h, src_h, dst_h, *refs):
        if with_den:
            (outm_h, outd_h, src_v, dst_v, gidx_v, midx_v,
             el_v, er_v, feat_v, exw_v, den_t, acc, sem) = refs
        else:
            (outm_h, src_v, dst_v, gidx_v, midx_v,
             el_v, er_v, feat_v, exw_v, acc, sem) = refs
        c = lax.axis_index("c")
        s = lax.axis_index("s")
        zero16 = jnp.zeros((16,), jnp.float32)
        lane = lax.iota(jnp.int32, 16)
        head_mask = lane < 8
        rot_idx = (lane + 8) & 15
        cc = c * NH

        zsrc = den_t if with_den else feat_v
        @pl.loop(0, 40)
        def _(r):
            for j in range(8):
                zsrc[r, pl.ds(j * 16, 16)] = zero16
        if with_den:
            @pl.loop(40, DROWS_T)
            def _(r):
                for j in range(8):
                    den_t[r, pl.ds(j * 16, 16)] = zero16

        per_tile = acc_rows // SC_TILES
        row0 = s * per_tile
        for k in range(per_tile // 40):
            pltpu.sync_copy(zsrc.at[pl.ds(0, 40)],
                            acc.at[pl.ds(row0 + k * 40, 40)])
        if per_tile % 40:
            pltpu.sync_copy(zsrc.at[pl.ds(0, per_tile % 40)],
                            acc.at[pl.ds(row0 + (per_tile // 40) * 40,
                                         per_tile % 40)])
        plsc.subcore_barrier()

        edge0 = s * EDGES_PER_TILE

        @pl.loop(0, WINDOWS)
        def _(w):
            base = edge0 + w * W_EDGES
            cp1 = pltpu.async_copy(src_h.at[pl.ds(base, W_EDGES)], src_v, sem)
            cp2 = pltpu.async_copy(dst_h.at[pl.ds(base, W_EDGES)], dst_v, sem)
            cp1.wait()
            cp2.wait()

            @pl.loop(0, W_EDGES // 16)
            def _(j):
                sl = pl.ds(j * 16, 16)
                gidx_v[sl] = src_v[sl] + q * N
                nloc = dst_v[sl] - cc
                inr = (nloc >= 0) & (nloc < NH)
                midx_v[sl] = jnp.where(inr, nloc,
                                       NH + 120 + (nloc & 63))

            g1 = pltpu.async_copy(feat_h.at[gidx_v], feat_v, sem)
            g2 = pltpu.async_copy(ea_h.at[src_v], el_v, sem)
            g3 = pltpu.async_copy(ea_h.at[dst_v], er_v, sem)
            g2.wait()
            g3.wait()

            @pl.loop(0, W_EDGES // 16)
            def _(jc):
                dchunk = dst_v[pl.ds(jc * 16, 16)] - cc
                for k in range(16):
                    i = jc * 16 + k
                    e = el_v[i, pl.ds(0, 16)] + er_v[i, pl.ds(16, 16)]
                    e = jnp.where(e >= 0.0, e, 0.2 * e)
                    ex = jnp.exp(jnp.minimum(e, 60.0))
                    exw_v[i, pl.ds(0, 16)] = ex
                    if with_den:
                        exm = jnp.where(head_mask, ex, 0.0)
                        exr = _vgather16(exm, rot_idx)
                        d = dchunk[k]
                        inr = (d >= 0) & (d < NH)
                        dsub = d & 15
                        r = jnp.where(inr, lax.shift_right_arithmetic(d, 4),
                                      DROWS_T - 1)
                        off = jnp.where(inr,
                                        jnp.where(dsub == 15, 112,
                                                  dsub * 8), 0)
                        exu = jnp.where(dsub == 15, exr, exm)
                        den_t[r, pl.ds(off, 16)] = (
                            den_t[r, pl.ds(off, 16)] + exu)

            g1.wait()

            @pl.loop(0, W_EDGES)
            def _(i):
                ex = exw_v[i, pl.ds(0, 16)]
                for j in range(4):
                    bsel = jnp.full((16,), 4 * q + j, jnp.int32)
                    b = _vgather16(ex, bsel)
                    feat_v[i, pl.ds(j * 32, 16)] = (
                        feat_v[i, pl.ds(j * 32, 16)] * b)
                    feat_v[i, pl.ds(j * 32 + 16, 16)] = (
                        feat_v[i, pl.ds(j * 32 + 16, 16)] * b)

            pltpu.sync_copy(feat_v, acc.at[midx_v], add=True)

        plsc.subcore_barrier()

        if with_den:
            for k in range(DROWS_T // W_EDGES):
                @pl.loop(0, W_EDGES // 16)
                def _(j):
                    sl = pl.ds(j * 16, 16)
                    midx_v[sl] = lane + (MROWS + k * W_EDGES + j * 16)
                pltpu.sync_copy(den_t.at[pl.ds(k * W_EDGES, W_EDGES)],
                                acc.at[midx_v], add=True)
            plsc.subcore_barrier()

        mrow0 = s * (MSG_ROWS // SC_TILES)
        for k in range(MSG_ROWS // SC_TILES // W_EDGES):
            rows = pl.ds(mrow0 + k * W_EDGES, W_EDGES)
            pltpu.sync_copy(acc.at[rows], el_v)
            pltpu.sync_copy(el_v, outm_h.at[c, rows])
        if with_den:
            @pl.when(s < 8)
            def _():
                rows = pl.ds(s * 40, 40)
                pltpu.sync_copy(acc.at[pl.ds(MROWS + s * 40, 40)],
                                er_v.at[pl.ds(0, 40)])
                pltpu.sync_copy(er_v.at[pl.ds(0, 40)], outd_h.at[c, rows])

    return edge_kernel(feat_stacked, earr, src, dst)


def _stage_c_body(ctx, m00, m01, d0, m10, m11, d1, m20, m21, d2,
                  pj0, pb0, pj1, pb1, pj2, pb2,
                  gwh0, gwc0, gb0, gwh1, gwc1, gb1, gwh2, gwc2, gb2,
                  sw1, sb1, sw2, cw1, cb1, cw2, cb2, dexp,
                  logits_o):
    ctxv = ctx[...]
    de = dexp[...]

    zs = []
    for ma, mb, dn, pj, pb, gwh, gwc, gb in (
            (m00, m01, d0, pj0, pb0, gwh0, gwc0, gb0),
            (m10, m11, d1, pj1, pb1, gwh1, gwc1, gb1),
            (m20, m21, d2, pj2, pb2, gwh2, gwc2, gb2)):
        num = jnp.concatenate([ma[...], mb[...]], axis=1)
        den = _dot(dn[...], de)
        hg = num / (den + 1e-9)
        hg = jnp.where(hg > 0.0, hg, jnp.exp(jnp.minimum(hg, 0.0)) - 1.0)
        hp = _dot(hg, pj[...]) + pb[...]
        gate = jax.nn.sigmoid(_dot(hp, gwh[...]) + _dot(ctxv, gwc[...]) + gb[...])
        zs.append(gate * hp + (1.0 - gate) * ctxv)

    ws = [_dot(jnp.tanh(_dot(z, sw1[...]) + sb1[...]), sw2[...]) for z in zs]
    m = jnp.maximum(jnp.maximum(ws[0], ws[1]), ws[2])
    es = [jnp.exp(wv - m) for wv in ws]
    tot = es[0] + es[1] + es[2]
    hf = (es[0] * zs[0] + es[1] * zs[1] + es[2] * zs[2]) / tot

    hid = jnp.maximum(_dot(hf, cw1[...]) + cb1[...], 0.0)
    logits_o[...] = _dot(hid, cw2[...]) + cb2[...]


def _stage_c(ctx, outs, params):
    bn = 1000
    grid = (N // bn,)
    row = lambda i: (i, 0)
    fix = lambda i: (0, 0)

    def rspec(c):
        return pl.BlockSpec((bn, c), row)

    def wspec(shape):
        return pl.BlockSpec(shape, fix)

    p = params
    gats = p['gat']

    dexp = jnp.repeat(jnp.eye(HEADS, dtype=jnp.float32), DH, axis=1)

    in_specs = [rspec(H)] + [rspec(HALF), rspec(HALF), rspec(HEADS)] * 3 + [
        wspec((H, H)), wspec((1, H)),
        wspec((H, H)), wspec((1, H)),
        wspec((H, H)), wspec((1, H)),
        wspec((H, H)), wspec((H, H)), wspec((1, H)),
        wspec((H, H)), wspec((H, H)), wspec((1, H)),
        wspec((H, H)), wspec((H, H)), wspec((1, H)),
        wspec((H, H)), wspec((1, H)), wspec((H, 1)),
        wspec((H, H)), wspec((1, H)), wspec((H, CLS)), wspec((1, CLS)),
        wspec((HEADS, H)),
    ]

    args = [ctx]
    for m0, m1, dn in outs:
        args.append(m0)
        args.append(m1)
        args.append(dn)
    for g in gats:
        args.append(g['proj_w'])
        args.append(g['proj_b'][None, :])
    for g in gats:
        args.append(g['gate_w'][:H])
        args.append(g['gate_w'][H:])
        args.append(g['gate_b'][None, :])
    args += [
        p['sem']['w1'], p['sem']['b1'][None, :], p['sem']['w2'],
        p['cls']['w1'], p['cls']['b1'][None, :],
        p['cls']['w2'], p['cls']['b2'][None, :],
        dexp,
    ]

    return pl.pallas_call(
        _stage_c_body,
        grid=grid,
        in_specs=in_specs,
        out_specs=pl.BlockSpec((bn, CLS), row),
        out_shape=jax.ShapeDtypeStruct((N, CLS), jnp.float32),
    )(*args)


def _spread_attn(a):
    eye = jnp.eye(HEADS, 16, dtype=jnp.float32)
    return (a[:, :, None] * eye[:, None, :]).reshape(H, 16)


def kernel(user_feats, post_content, parent_comment,
           edge_ucu, edge_comment, edge_publish, params):
    gsum = jnp.repeat(jnp.eye(MHA_HEADS, dtype=jnp.float32), MHA_DH, axis=0)
    gexp = gsum.T
    spreads = [(_spread_attn(g['al']), _spread_attn(g['ar']))
               for g in params['gat']]

    a_out = _stage_a(user_feats, post_content, parent_comment, params,
                     gsum, gexp, spreads)
    ctx = a_out[0]
    path_feats = []
    for i in range(3):
        fl, fr, ea = a_out[2 + 3 * i: 5 + 3 * i]
        path_feats.append((jnp.concatenate([fl, fr], axis=0), ea))

    edges = (edge_ucu, edge_comment, edge_publish)
    outs = []
    for (feat2, ea), ei in zip(path_feats, edges):
        outm0, outd = _gat_edges_sc(feat2, ea, ei[0], ei[1], 0, True)
        res1 = _gat_edges_sc(feat2, ea, ei[0], ei[1], 1, False)
        outm1 = res1[0] if isinstance(res1, (tuple, list)) else res1
        nh = N // 2
        num_l = jnp.concatenate([outm0[0, :nh], outm0[1, :nh]], axis=0)
        num_r = jnp.concatenate([outm1[0, :nh], outm1[1, :nh]], axis=0)
        den8 = jnp.concatenate(
            [outd[0].reshape(DROWS_T * 16, HEADS)[:nh],
             outd[1].reshape(DROWS_T * 16, HEADS)[:nh]], axis=0)
        outs.append((num_l, num_r, den8))

    return _stage_c(ctx, outs, params)
